# baseline (speedup 1.0000x reference)
import functools

import jax
import jax.numpy as jnp
from jax.experimental import pallas as pl
from jax.experimental.pallas import tpu as pltpu

_TOPK = 2


def _moe_kernel(x_ref, g_ref, w1_ref, w2_ref, o_ref, cw_ref):
    e = pl.program_id(0)
    f = pl.program_id(1)

    @pl.when((e == 0) & (f == 0))
    def _router():
        x = x_ref[...]
        logits = jnp.dot(x, g_ref[...], preferred_element_type=jnp.float32)
        m = jnp.max(logits, axis=-1, keepdims=True)
        ex = jnp.exp(logits - m)
        p = ex / jnp.sum(ex, axis=-1, keepdims=True)
        lane = jax.lax.broadcasted_iota(jnp.int32, p.shape, 1)
        m1 = jnp.max(p, axis=-1, keepdims=True)
        i1 = jnp.min(jnp.where(p == m1, lane, p.shape[1]), axis=-1, keepdims=True)
        p2 = jnp.where(lane == i1, -1.0, p)
        m2 = jnp.max(p2, axis=-1, keepdims=True)
        i2 = jnp.min(jnp.where(p2 == m2, lane, p.shape[1]), axis=-1, keepdims=True)
        cw = jnp.where(lane == i1, m1, jnp.where(lane == i2, m2, 0.0))
        cw_ref[...] = cw / (m1 + m2)
        o_ref[...] = jnp.zeros_like(o_ref)

    cw = cw_ref[...]
    lane = jax.lax.broadcasted_iota(jnp.int32, cw.shape, 1)
    scale = jnp.sum(jnp.where(lane == e, cw, 0.0), axis=-1, keepdims=True)

    x = x_ref[...]
    h = jnp.dot(x, w1_ref[0], preferred_element_type=jnp.float32)
    h = h * jax.nn.sigmoid(h)
    h = h * scale
    o_ref[...] += jnp.dot(h, w2_ref[0], preferred_element_type=jnp.float32)


@jax.jit
def kernel(hidden_states, gate_w, w1, w2):
    T, D = hidden_states.shape
    E = gate_w.shape[1]
    F = w1.shape[2]
    FB = 1024
    nf = F // FB

    grid = (E, nf)
    return pl.pallas_call(
        _moe_kernel,
        grid=grid,
        in_specs=[
            pl.BlockSpec((T, D), lambda e, f: (0, 0)),
            pl.BlockSpec((D, E), lambda e, f: (0, 0)),
            pl.BlockSpec((1, D, FB), lambda e, f: (e, 0, f)),
            pl.BlockSpec((1, FB, D), lambda e, f: (e, f, 0)),
        ],
        out_specs=pl.BlockSpec((T, D), lambda e, f: (0, 0)),
        out_shape=jax.ShapeDtypeStruct((T, D), jnp.float32),
        scratch_shapes=[pltpu.VMEM((T, E), jnp.float32)],
        compiler_params=pltpu.CompilerParams(
            dimension_semantics=("arbitrary", "arbitrary"),
        ),
    )(hidden_states, gate_w, w1, w2)


# speedup vs baseline: 1.0142x; 1.0142x over previous
import functools

import jax
import jax.numpy as jnp
from jax.experimental import pallas as pl
from jax.experimental.pallas import tpu as pltpu

_TOPK = 2


def _moe_kernel(x_ref, g_ref, w1_ref, w2_ref, o_ref, cw_ref):
    e = pl.program_id(0)
    f = pl.program_id(1)

    @pl.when((e == 0) & (f == 0))
    def _router():
        x = x_ref[...]
        logits = jnp.dot(x, g_ref[...], preferred_element_type=jnp.float32)
        m = jnp.max(logits, axis=-1, keepdims=True)
        ex = jnp.exp(logits - m)
        p = ex / jnp.sum(ex, axis=-1, keepdims=True)
        lane = jax.lax.broadcasted_iota(jnp.int32, p.shape, 1)
        m1 = jnp.max(p, axis=-1, keepdims=True)
        i1 = jnp.min(jnp.where(p == m1, lane, p.shape[1]), axis=-1, keepdims=True)
        p2 = jnp.where(lane == i1, -1.0, p)
        m2 = jnp.max(p2, axis=-1, keepdims=True)
        i2 = jnp.min(jnp.where(p2 == m2, lane, p.shape[1]), axis=-1, keepdims=True)
        cw = jnp.where(lane == i1, m1, jnp.where(lane == i2, m2, 0.0))
        cw_ref[...] = cw / (m1 + m2)
        o_ref[...] = jnp.zeros_like(o_ref)

    cw = cw_ref[...]
    lane = jax.lax.broadcasted_iota(jnp.int32, cw.shape, 1)
    scale = jnp.sum(jnp.where(lane == e, cw, 0.0), axis=-1, keepdims=True)

    x = x_ref[...].astype(jnp.bfloat16)
    h = jnp.dot(x, w1_ref[0].astype(jnp.bfloat16),
                preferred_element_type=jnp.float32)
    h = h * jax.nn.sigmoid(h)
    h = h * scale
    o_ref[...] += jnp.dot(h.astype(jnp.bfloat16),
                          w2_ref[0].astype(jnp.bfloat16),
                          preferred_element_type=jnp.float32)


@jax.jit
def kernel(hidden_states, gate_w, w1, w2):
    T, D = hidden_states.shape
    E = gate_w.shape[1]
    F = w1.shape[2]
    FB = 512
    nf = F // FB

    grid = (E, nf)
    return pl.pallas_call(
        _moe_kernel,
        grid=grid,
        in_specs=[
            pl.BlockSpec((T, D), lambda e, f: (0, 0)),
            pl.BlockSpec((D, E), lambda e, f: (0, 0)),
            pl.BlockSpec((1, D, FB), lambda e, f: (e, 0, f)),
            pl.BlockSpec((1, FB, D), lambda e, f: (e, f, 0)),
        ],
        out_specs=pl.BlockSpec((T, D), lambda e, f: (0, 0)),
        out_shape=jax.ShapeDtypeStruct((T, D), jnp.float32),
        scratch_shapes=[pltpu.VMEM((T, E), jnp.float32)],
        compiler_params=pltpu.CompilerParams(
            dimension_semantics=("arbitrary", "arbitrary"),
        ),
    )(hidden_states, gate_w, w1, w2)
